# Initial kernel scaffold; baseline (speedup 1.0000x reference)
import jax
import jax.numpy as jnp
from jax.experimental import pallas as pl


def kernel(doc_codes, sub_weights):
    raise NotImplementedError("write your pallas kernel here")


# speedup vs baseline: 20.8459x; 20.8459x over previous
import jax
import jax.numpy as jnp
from jax import lax
from jax.experimental import pallas as pl
from jax.experimental.pallas import tpu as pltpu
from jax.experimental.pallas import tpu_sc as plsc

M = 48
K = 256
DSUB = 16
B = 16384

NC = 2
NS = 16
NW = NC * NS
ROWS = B * M
RPW = ROWS // NW
CHUNK = 3072
ISLICE = 128
NSTREAM = CHUNK // ISLICE
NCHUNK = RPW // CHUNK


def _gather_body(codes_hbm, table_hbm, out_hbm, idx_v, rows_v, sem):
    wid = lax.axis_index("s") * NC + lax.axis_index("c")
    base = pl.multiple_of(wid * RPW, RPW)
    pltpu.sync_copy(codes_hbm.at[pl.ds(base, RPW)], idx_v)

    lane = lax.iota(jnp.int32, 16)

    def add_offs(j, carry):
        start = pl.multiple_of(j * 16, 16)
        m0 = lax.rem(start, M)
        idx_v[pl.ds(start, 16)] = idx_v[pl.ds(start, 16)] + (m0 + lane) * K
        return carry

    lax.fori_loop(0, RPW // 16, add_offs, 0)

    for ci in range(NCHUNK):
        cb = ci * CHUNK
        copies = [
            pltpu.async_copy(
                table_hbm.at[idx_v.at[pl.ds(cb + s * ISLICE, ISLICE)]],
                rows_v.at[pl.ds(s * ISLICE, ISLICE)],
                sem,
            )
            for s in range(NSTREAM)
        ]
        for cp in copies:
            cp.wait()
        pltpu.sync_copy(rows_v, out_hbm.at[pl.ds(base + cb, CHUNK)])


@jax.jit
def _impl(doc_codes, sub_weights):
    codes = doc_codes.astype(jnp.int32).reshape(ROWS)
    table = sub_weights.reshape(M * K, DSUB)
    mesh = plsc.VectorSubcoreMesh(core_axis_name="c", subcore_axis_name="s")
    out = pl.kernel(
        _gather_body,
        out_type=jax.ShapeDtypeStruct((ROWS, DSUB), jnp.float32),
        mesh=mesh,
        compiler_params=pltpu.CompilerParams(use_tc_tiling_on_sc=False),
        scratch_types=[
            pltpu.VMEM((RPW,), jnp.int32),
            pltpu.VMEM((CHUNK, DSUB), jnp.float32),
            pltpu.SemaphoreType.DMA,
        ],
    )(codes, table)
    return out.reshape(B, M * DSUB)


def kernel(doc_codes, sub_weights):
    return _impl(doc_codes, sub_weights)


# speedup vs baseline: 21.8809x; 1.0497x over previous
import jax
import jax.numpy as jnp
from jax import lax
from jax.experimental import pallas as pl
from jax.experimental.pallas import tpu as pltpu
from jax.experimental.pallas import tpu_sc as plsc

M = 48
K = 256
DSUB = 16
B = 16384

NC = 2
NS = 16
NW = NC * NS
ROWS = B * M
RPW = ROWS // NW
CHUNK = 3072
ISLICE = 128
NSTREAM = CHUNK // ISLICE
NCHUNK = RPW // CHUNK


def _gather_body(codes_hbm, table_hbm, out_hbm, idx_v, rows0, rows1, sem_g,
                 sem_o0, sem_o1):
    wid = lax.axis_index("s") * NC + lax.axis_index("c")
    base = pl.multiple_of(wid * RPW, RPW)
    pltpu.sync_copy(codes_hbm.at[pl.ds(base, RPW)], idx_v)

    lane = lax.iota(jnp.int32, 16)
    offs = [(r * 16 + lane) * K for r in range(3)]

    def add_offs(g, carry):
        s0 = pl.multiple_of(g * 48, 16)
        for r in range(3):
            sl = pl.ds(pl.multiple_of(s0 + r * 16, 16), 16)
            idx_v[sl] = idx_v[sl] + offs[r]
        return carry

    lax.fori_loop(0, RPW // 48, add_offs, 0)

    rows = (rows0, rows1)
    sem_o = (sem_o0, sem_o1)
    out_cp = [None, None]
    for ci in range(NCHUNK):
        b = ci & 1
        if out_cp[b] is not None:
            out_cp[b].wait()
        cb = ci * CHUNK
        copies = [
            pltpu.async_copy(
                table_hbm.at[idx_v.at[pl.ds(cb + s * ISLICE, ISLICE)]],
                rows[b].at[pl.ds(s * ISLICE, ISLICE)],
                sem_g,
            )
            for s in range(NSTREAM)
        ]
        for cp in copies:
            cp.wait()
        out_cp[b] = pltpu.async_copy(
            rows[b], out_hbm.at[pl.ds(base + cb, CHUNK)], sem_o[b]
        )
    for b in range(2):
        if out_cp[b] is not None:
            out_cp[b].wait()


@jax.jit
def _impl(doc_codes, sub_weights):
    codes = doc_codes.astype(jnp.int32).reshape(ROWS)
    table = sub_weights.reshape(M * K, DSUB)
    mesh = plsc.VectorSubcoreMesh(core_axis_name="c", subcore_axis_name="s")
    out = pl.kernel(
        _gather_body,
        out_type=jax.ShapeDtypeStruct((ROWS, DSUB), jnp.float32),
        mesh=mesh,
        compiler_params=pltpu.CompilerParams(use_tc_tiling_on_sc=False),
        scratch_types=[
            pltpu.VMEM((RPW,), jnp.int32),
            pltpu.VMEM((CHUNK, DSUB), jnp.float32),
            pltpu.VMEM((CHUNK, DSUB), jnp.float32),
            pltpu.SemaphoreType.DMA,
            pltpu.SemaphoreType.DMA,
            pltpu.SemaphoreType.DMA,
        ],
    )(codes, table)
    return out.reshape(B, M * DSUB)


def kernel(doc_codes, sub_weights):
    return _impl(doc_codes, sub_weights)


# speedup vs baseline: 26.6514x; 1.2180x over previous
import jax
import jax.numpy as jnp
from jax import lax
from jax.experimental import pallas as pl
from jax.experimental.pallas import tpu as pltpu
from jax.experimental.pallas import tpu_sc as plsc

M = 48
K = 256
DSUB = 16
B = 16384

NC = 2
NS = 16
NW = NC * NS
ROWS = B * M
RPW = ROWS // NW
CHUNK = 1536
ISLICE = 128
NSTREAM = CHUNK // ISLICE
NCHUNK = RPW // CHUNK


def _gather_body(codes_hbm, table_hbm, out_hbm, idx_v, rows0, rows1, tab_s,
                 sem_g, sem_o0, sem_o1):
    wid = lax.axis_index("s") * NC + lax.axis_index("c")
    base = pl.multiple_of(wid * RPW, RPW)

    @pl.when(lax.axis_index("s") == 0)
    def _stage():
        pltpu.sync_copy(table_hbm, tab_s)

    plsc.subcore_barrier()

    pltpu.sync_copy(codes_hbm.at[pl.ds(base, RPW)], idx_v)

    lane = lax.iota(jnp.int32, 16)
    offs = [(r * 16 + lane) * K for r in range(3)]

    def add_offs(g, carry):
        s0 = pl.multiple_of(g * 48, 16)
        for r in range(3):
            sl = pl.ds(pl.multiple_of(s0 + r * 16, 16), 16)
            idx_v[sl] = idx_v[sl] + offs[r]
        return carry

    lax.fori_loop(0, RPW // 48, add_offs, 0)

    rows = (rows0, rows1)
    sem_o = (sem_o0, sem_o1)
    out_cp = [None, None]
    for ci in range(NCHUNK):
        b = ci & 1
        if out_cp[b] is not None:
            out_cp[b].wait()
        cb = ci * CHUNK
        copies = [
            pltpu.async_copy(
                tab_s.at[idx_v.at[pl.ds(cb + s * ISLICE, ISLICE)]],
                rows[b].at[pl.ds(s * ISLICE, ISLICE)],
                sem_g,
            )
            for s in range(NSTREAM)
        ]
        for cp in copies:
            cp.wait()
        out_cp[b] = pltpu.async_copy(
            rows[b], out_hbm.at[pl.ds(base + cb, CHUNK)], sem_o[b]
        )
    for b in range(2):
        if out_cp[b] is not None:
            out_cp[b].wait()


@jax.jit
def _impl(doc_codes, sub_weights):
    codes = doc_codes.astype(jnp.int32).reshape(ROWS)
    table = sub_weights.reshape(M * K, DSUB)
    mesh = plsc.VectorSubcoreMesh(core_axis_name="c", subcore_axis_name="s")
    out = pl.kernel(
        _gather_body,
        out_type=jax.ShapeDtypeStruct((ROWS, DSUB), jnp.float32),
        mesh=mesh,
        compiler_params=pltpu.CompilerParams(use_tc_tiling_on_sc=False),
        scratch_types=[
            pltpu.VMEM((RPW,), jnp.int32),
            pltpu.VMEM((CHUNK, DSUB), jnp.float32),
            pltpu.VMEM((CHUNK, DSUB), jnp.float32),
            pltpu.VMEM_SHARED((M * K, DSUB), jnp.float32),
            pltpu.SemaphoreType.DMA,
            pltpu.SemaphoreType.DMA,
            pltpu.SemaphoreType.DMA,
        ],
    )(codes, table)
    return out.reshape(B, M * DSUB)


def kernel(doc_codes, sub_weights):
    return _impl(doc_codes, sub_weights)
